# Initial kernel scaffold; baseline (speedup 1.0000x reference)
import jax
import jax.numpy as jnp
from jax.experimental import pallas as pl


def kernel(sinogram):
    raise NotImplementedError("write your pallas kernel here")


# speedup vs baseline: 10.7862x; 10.7862x over previous
import functools

import numpy as np
import jax
import jax.numpy as jnp
from jax import lax
from jax.experimental import pallas as pl
from jax.experimental.pallas import tpu as pltpu
from jax.experimental.pallas import tpu_sc as plsc

_B = 32
_NSEL = 36
_NPAD = 48
_TBL = 1040


def _cl(v, lo, hi):
    return max(lo, min(hi, v))


def _tc_body(x_ref, bin_ref, best_ref):
    x = x_ref[0, 0]
    mn = jnp.min(x)
    mx = jnp.max(x)
    norm = jnp.clip((x - mn) / (mx - mn + 1e-6), 0.0, 1.0)
    bi = jnp.clip((norm * 63.0).astype(jnp.int32), 0, 63)
    bin_ref[0] = bi
    bf = bi.astype(jnp.float32)
    trows = jnp.concatenate([bf[64 * k:64 * k + 32] for k in range(6)], axis=0)
    tsel = jnp.concatenate(
        [trows[:, 64 * l:64 * l + 32] for l in range(6)], axis=1)
    r6 = lax.broadcasted_iota(jnp.int32, (6, 192), 0)
    j192 = lax.broadcasted_iota(jnp.int32, (6, 192), 1)
    srow = (j192 // 32 == r6).astype(jnp.float32)
    jc = lax.broadcasted_iota(jnp.int32, (192, 6), 0)
    c6 = lax.broadcasted_iota(jnp.int32, (192, 6), 1)
    scol = (jc // 32 == c6).astype(jnp.float32)
    kk = lax.broadcasted_iota(jnp.int32, (6, 6), 0)
    ll = lax.broadcasted_iota(jnp.int32, (6, 6), 1)
    best_d = None
    best_p = None
    for dr in range(-2, 3):
        rblocks = [_cl(2 * k + dr, 0, 11) for k in range(6)]
        rsel = jnp.concatenate(
            [bf[32 * rb:32 * rb + 32] for rb in rblocks], axis=0)
        for dc in range(-2, 3):
            cblocks = [_cl(2 * l + dc, 0, 11) for l in range(6)]
            csel = jnp.concatenate(
                [rsel[:, 32 * cb:32 * cb + 32] for cb in cblocks], axis=1)
            d = jnp.abs(tsel - csel)
            dsum = jnp.dot(srow, d, preferred_element_type=jnp.float32)
            dist = jnp.dot(dsum, scol, preferred_element_type=jnp.float32)
            if dr == 0:
                rhit = kk >= 0
            elif dr < 0:
                rhit = kk == 0
            else:
                rhit = kk < 0
            if dc == 0:
                chit = ll >= 0
            elif dc < 0:
                chit = ll == 0
            else:
                chit = ll < 0
            mask = jnp.where(rhit & chit, 1e9, 0.0).astype(jnp.float32)
            dist = dist + mask
            bpj = (jnp.clip(2 * kk + dr, 0, 11) * 12
                   + jnp.clip(2 * ll + dc, 0, 11))
            if best_d is None:
                best_d, best_p = dist, bpj
            else:
                upd = dist < best_d
                best_p = jnp.where(upd, bpj, best_p)
                best_d = jnp.minimum(best_d, dist)
    b = pl.program_id(0)
    best_ref[0] = best_p + b * 144


@functools.cache
def _make_sc_mi():
    mesh = plsc.VectorSubcoreMesh(core_axis_name="c", subcore_axis_name="s")

    @functools.partial(
        pl.kernel,
        mesh=mesh,
        compiler_params=pltpu.CompilerParams(needs_layout_passes=False),
        out_type=jax.ShapeDtypeStruct((_B * _NPAD,), jnp.float32),
        scratch_types=[
            pltpu.VMEM((_NPAD,), jnp.int32),
            pltpu.VMEM((_NPAD,), jnp.int32),
            pltpu.VMEM((32,), jnp.int32),
            pltpu.VMEM((32, 1024), jnp.int32),
            pltpu.VMEM((16384,), jnp.int32),
            pltpu.VMEM((65536,), jnp.int32),
            pltpu.VMEM((1024,), jnp.int32),
            pltpu.VMEM((1024,), jnp.int32),
            pltpu.VMEM((_TBL,), jnp.float32),
            pltpu.VMEM((_TBL,), jnp.float32),
            pltpu.VMEM((_NPAD,), jnp.float32),
            pltpu.SemaphoreType.DMA,
        ],
    )
    def sc_mi(ftab_h, gtab_h, tgt_h, best_h, table_h, out_h,
              ids_t, ids_b, idx_buf, rows_v, jidx_buf, histj, hx, hy,
              ftab_v, gtab_v, out_v, sem):
        wid = lax.axis_index("s") * 2 + lax.axis_index("c")
        iota = lax.iota(jnp.int32, 16)
        ones = jnp.ones((16,), jnp.int32)
        zi = jnp.zeros((16,), jnp.int32)
        zf = jnp.zeros((16,), jnp.float32)
        pltpu.sync_copy(ftab_h, ftab_v)
        pltpu.sync_copy(gtab_h, gtab_v)
        pltpu.sync_copy(tgt_h.at[wid], ids_t)
        pltpu.sync_copy(best_h.at[wid], ids_b)

        def zj(i, c):
            b0 = i * 64
            histj[pl.ds(b0, 16)] = zi
            histj[pl.ds(b0 + 16, 16)] = zi
            histj[pl.ds(b0 + 32, 16)] = zi
            histj[pl.ds(b0 + 48, 16)] = zi
            return c
        lax.fori_loop(0, 1024, zj, 0)

        def zxy(i, c):
            hx[pl.ds(i * 16, 16)] = zi
            hy[pl.ds(i * 16, 16)] = zi
            return c
        lax.fori_loop(0, 64, zxy, 0)

        for g in range(3):
            idx_buf[pl.ds(0, 16)] = ids_t[pl.ds(g * 16, 16)]
            idx_buf[pl.ds(16, 16)] = ids_b[pl.ds(g * 16, 16)]
            pltpu.async_copy(table_h.at[idx_buf], rows_v, sem).wait()

            def p1(it, c):
                for u in range(4):
                    p = it * 4 + u
                    pp = zi + p
                    xr = plsc.load_gather(rows_v, [iota, pp])
                    yr = plsc.load_gather(rows_v, [iota + 16, pp])
                    jx = (xr * 64 + yr) * 16 + iota
                    jidx_buf[pl.ds(p * 16, 16)] = jx
                    plsc.addupdate_scatter(histj, [jx], ones)
                    plsc.addupdate_scatter(hx, [xr * 16 + iota], ones)
                    plsc.addupdate_scatter(hy, [yr * 16 + iota], ones)
                return c
            lax.fori_loop(0, 256, p1, 0)

            def p2(it, acc):
                for u in range(4):
                    p = it * 4 + u
                    jx = jidx_buf[pl.ds(p * 16, 16)]
                    cnt = plsc.load_gather(histj, [jx])
                    acc = acc + plsc.load_gather(gtab_v, [cnt])
                return acc
            accj = lax.fori_loop(0, 256, p2, zf)

            def p3(it, c):
                for u in range(4):
                    p = it * 4 + u
                    jx = jidx_buf[pl.ds(p * 16, 16)]
                    plsc.store_scatter(histj, [jx], zi)
                return c
            lax.fori_loop(0, 256, p3, 0)

            def pxy(a, accs):
                ax, ay = accs
                cx = hx[pl.ds(a * 16, 16)]
                cy = hy[pl.ds(a * 16, 16)]
                ax = ax + plsc.load_gather(ftab_v, [cx])
                ay = ay + plsc.load_gather(ftab_v, [cy])
                hx[pl.ds(a * 16, 16)] = zi
                hy[pl.ds(a * 16, 16)] = zi
                return (ax, ay)
            accx, accy = lax.fori_loop(0, 64, pxy, (zf, zf))

            mi = accj - accx - accy
            v = mi + 1.0
            bits = plsc.bitcast(v, jnp.int32)
            e = (bits >> 23) - 127
            m = plsc.bitcast((bits & 0x7FFFFF) | 0x3F800000, jnp.float32)
            t = (m - 1.0) / (m + 1.0)
            t2 = t * t
            ln_m = t * (2.0 + t2 * (0.6666666666 + t2 * (
                0.4 + t2 * (0.2857142857 + t2 * 0.2222222222))))
            res = e.astype(jnp.float32) * 0.6931471805599453 + ln_m
            out_v[pl.ds(g * 16, 16)] = res

        pltpu.sync_copy(out_v, out_h.at[pl.ds(wid * _NPAD, _NPAD)])

    return sc_mi


def _tables():
    c = np.arange(_TBL, dtype=np.float64)
    p = c / 1024.0
    f = p * np.log(p + 1e-8)
    g = np.zeros(_TBL, np.float64)
    g[1:] = f[1:] / c[1:]
    return f.astype(np.float32), g.astype(np.float32)


_FTAB, _GTAB = _tables()


def _tgt_ids():
    ids = np.zeros((_B, _NPAD), np.int32)
    for b in range(_B):
        for t in range(_NSEL):
            ids[b, t] = b * 144 + (2 * (t // 6)) * 12 + 2 * (t % 6)
        ids[b, _NSEL:] = b * 144
    return ids


_TGT48 = _tgt_ids()


def kernel(sinogram):
    if sinogram.ndim == 3:
        sinogram = sinogram[:, None]
    binned, best = pl.pallas_call(
        _tc_body,
        grid=(_B,),
        in_specs=[pl.BlockSpec((1, 1, 384, 384), lambda b: (b, 0, 0, 0))],
        out_specs=[pl.BlockSpec((1, 384, 384), lambda b: (b, 0, 0)),
                   pl.BlockSpec((1, 6, 6), lambda b: (b, 0, 0))],
        out_shape=[jax.ShapeDtypeStruct((_B, 384, 384), jnp.int32),
                   jax.ShapeDtypeStruct((_B, 6, 6), jnp.int32)],
    )(sinogram)
    table = (binned.reshape(_B, 12, 32, 12, 32)
             .transpose(0, 1, 3, 2, 4).reshape(_B * 144, 1024))
    pad = jnp.broadcast_to(
        (jnp.arange(_B, dtype=jnp.int32) * 144)[:, None], (_B, _NPAD - _NSEL))
    best48 = jnp.concatenate([best.reshape(_B, _NSEL), pad], axis=1)
    mi = _make_sc_mi()(jnp.asarray(_FTAB), jnp.asarray(_GTAB),
                       jnp.asarray(_TGT48), best48, table)
    return mi.reshape(_B, _NPAD)[:, :_NSEL]


# speedup vs baseline: 11.0647x; 1.0258x over previous
import functools

import numpy as np
import jax
import jax.numpy as jnp
from jax import lax
from jax.experimental import pallas as pl
from jax.experimental.pallas import tpu as pltpu
from jax.experimental.pallas import tpu_sc as plsc

_B = 32
_NSEL = 36
_NPAD = 48
_TBL = 1040


def _cl(v, lo, hi):
    return max(lo, min(hi, v))


def _tc_body(x_ref, bin_ref, best_ref):
    x = x_ref[0, 0]
    mn = jnp.min(x)
    mx = jnp.max(x)
    norm = jnp.clip((x - mn) / (mx - mn + 1e-6), 0.0, 1.0)
    bi = jnp.clip((norm * 63.0).astype(jnp.int32), 0, 63)
    bin_ref[0] = bi
    bf = bi.astype(jnp.float32)
    trows = jnp.concatenate([bf[64 * k:64 * k + 32] for k in range(6)], axis=0)
    tsel = jnp.concatenate(
        [trows[:, 64 * l:64 * l + 32] for l in range(6)], axis=1)
    r6 = lax.broadcasted_iota(jnp.int32, (6, 192), 0)
    j192 = lax.broadcasted_iota(jnp.int32, (6, 192), 1)
    srow = (j192 // 32 == r6).astype(jnp.float32)
    jc = lax.broadcasted_iota(jnp.int32, (192, 6), 0)
    c6 = lax.broadcasted_iota(jnp.int32, (192, 6), 1)
    scol = (jc // 32 == c6).astype(jnp.float32)
    kk = lax.broadcasted_iota(jnp.int32, (6, 6), 0)
    ll = lax.broadcasted_iota(jnp.int32, (6, 6), 1)
    best_d = None
    best_p = None
    for dr in range(-2, 3):
        rblocks = [_cl(2 * k + dr, 0, 11) for k in range(6)]
        rsel = jnp.concatenate(
            [bf[32 * rb:32 * rb + 32] for rb in rblocks], axis=0)
        for dc in range(-2, 3):
            cblocks = [_cl(2 * l + dc, 0, 11) for l in range(6)]
            csel = jnp.concatenate(
                [rsel[:, 32 * cb:32 * cb + 32] for cb in cblocks], axis=1)
            d = jnp.abs(tsel - csel)
            dsum = jnp.dot(srow, d, preferred_element_type=jnp.float32)
            dist = jnp.dot(dsum, scol, preferred_element_type=jnp.float32)
            if dr == 0:
                rhit = kk >= 0
            elif dr < 0:
                rhit = kk == 0
            else:
                rhit = kk < 0
            if dc == 0:
                chit = ll >= 0
            elif dc < 0:
                chit = ll == 0
            else:
                chit = ll < 0
            mask = jnp.where(rhit & chit, 1e9, 0.0).astype(jnp.float32)
            dist = dist + mask
            bpj = (jnp.clip(2 * kk + dr, 0, 11) * 12
                   + jnp.clip(2 * ll + dc, 0, 11))
            if best_d is None:
                best_d, best_p = dist, bpj
            else:
                upd = dist < best_d
                best_p = jnp.where(upd, bpj, best_p)
                best_d = jnp.minimum(best_d, dist)
    b = pl.program_id(0)
    best_ref[0] = best_p + b * 144


@functools.cache
def _make_sc_mi():
    mesh = plsc.VectorSubcoreMesh(core_axis_name="c", subcore_axis_name="s")

    @functools.partial(
        pl.kernel,
        mesh=mesh,
        compiler_params=pltpu.CompilerParams(needs_layout_passes=False),
        out_type=jax.ShapeDtypeStruct((_B * _NPAD,), jnp.float32),
        scratch_types=[
            pltpu.VMEM((_NPAD,), jnp.int32),
            pltpu.VMEM((_NPAD,), jnp.int32),
            pltpu.VMEM((32,), jnp.int32),
            pltpu.VMEM((32, 1024), jnp.int32),
            pltpu.VMEM((16384,), jnp.int32),
            pltpu.VMEM((65536,), jnp.int32),
            pltpu.VMEM((1024,), jnp.int32),
            pltpu.VMEM((1024,), jnp.int32),
            pltpu.VMEM((_TBL,), jnp.float32),
            pltpu.VMEM((_NPAD,), jnp.float32),
            pltpu.SemaphoreType.DMA,
        ],
    )
    def sc_mi(ftab_h, tgt_h, best_h, table_h, out_h,
              ids_t, ids_b, idx_buf, rows_v, jidx_buf, histj, hx, hy,
              ftab_v, out_v, sem):
        wid = lax.axis_index("s") * 2 + lax.axis_index("c")
        iota = lax.iota(jnp.int32, 16)
        ones = jnp.ones((16,), jnp.int32)
        zi = jnp.zeros((16,), jnp.int32)
        zf = jnp.zeros((16,), jnp.float32)
        pltpu.sync_copy(ftab_h, ftab_v)
        pltpu.sync_copy(tgt_h.at[wid], ids_t)
        pltpu.sync_copy(best_h.at[wid], ids_b)

        def zj(i, c):
            b0 = i * 64
            histj[pl.ds(b0, 16)] = zi
            histj[pl.ds(b0 + 16, 16)] = zi
            histj[pl.ds(b0 + 32, 16)] = zi
            histj[pl.ds(b0 + 48, 16)] = zi
            return c
        lax.fori_loop(0, 1024, zj, 0)

        def zxy(i, c):
            hx[pl.ds(i * 16, 16)] = zi
            hy[pl.ds(i * 16, 16)] = zi
            return c
        lax.fori_loop(0, 64, zxy, 0)

        for g in range(3):
            idx_buf[pl.ds(0, 16)] = ids_t[pl.ds(g * 16, 16)]
            idx_buf[pl.ds(16, 16)] = ids_b[pl.ds(g * 16, 16)]
            pltpu.async_copy(table_h.at[idx_buf], rows_v, sem).wait()

            def p1(it, c):
                for u in range(8):
                    p = it * 8 + u
                    pp = zi + p
                    xr = plsc.load_gather(rows_v, [iota, pp])
                    yr = plsc.load_gather(rows_v, [iota + 16, pp])
                    jx = (xr * 64 + yr) * 16 + iota
                    jidx_buf[pl.ds(p * 16, 16)] = jx
                    plsc.addupdate_scatter(histj, [jx], ones)
                    plsc.addupdate_scatter(hx, [xr * 16 + iota], ones)
                    plsc.addupdate_scatter(hy, [yr * 16 + iota], ones)
                return c
            lax.fori_loop(0, 128, p1, 0)

            def p2(it, acc):
                for u in range(8):
                    p = it * 8 + u
                    jx = jidx_buf[pl.ds(p * 16, 16)]
                    cnt = plsc.load_gather(histj, [jx])
                    plsc.store_scatter(histj, [jx], zi)
                    acc = acc + plsc.load_gather(ftab_v, [cnt])
                return acc
            accj = lax.fori_loop(0, 128, p2, zf)

            def pxy(a, accs):
                ax, ay = accs
                cx = hx[pl.ds(a * 16, 16)]
                cy = hy[pl.ds(a * 16, 16)]
                ax = ax + plsc.load_gather(ftab_v, [cx])
                ay = ay + plsc.load_gather(ftab_v, [cy])
                hx[pl.ds(a * 16, 16)] = zi
                hy[pl.ds(a * 16, 16)] = zi
                return (ax, ay)
            accx, accy = lax.fori_loop(0, 64, pxy, (zf, zf))

            mi = accj - accx - accy
            v = mi + 1.0
            bits = plsc.bitcast(v, jnp.int32)
            e = (bits >> 23) - 127
            m = plsc.bitcast((bits & 0x7FFFFF) | 0x3F800000, jnp.float32)
            t = (m - 1.0) / (m + 1.0)
            t2 = t * t
            ln_m = t * (2.0 + t2 * (0.6666666666 + t2 * (
                0.4 + t2 * (0.2857142857 + t2 * 0.2222222222))))
            res = e.astype(jnp.float32) * 0.6931471805599453 + ln_m
            out_v[pl.ds(g * 16, 16)] = res

        pltpu.sync_copy(out_v, out_h.at[pl.ds(wid * _NPAD, _NPAD)])

    return sc_mi


def _tables():
    c = np.arange(_TBL, dtype=np.float64)
    p = c / 1024.0
    f = p * np.log(p + 1e-8)
    return f.astype(np.float32)


_FTAB = _tables()


def _tgt_ids():
    ids = np.zeros((_B, _NPAD), np.int32)
    for b in range(_B):
        for t in range(_NSEL):
            ids[b, t] = b * 144 + (2 * (t // 6)) * 12 + 2 * (t % 6)
        ids[b, _NSEL:] = b * 144 + np.arange(_NPAD - _NSEL)
    return ids


_TGT48 = _tgt_ids()


def kernel(sinogram):
    if sinogram.ndim == 3:
        sinogram = sinogram[:, None]
    binned, best = pl.pallas_call(
        _tc_body,
        grid=(_B,),
        in_specs=[pl.BlockSpec((1, 1, 384, 384), lambda b: (b, 0, 0, 0))],
        out_specs=[pl.BlockSpec((1, 384, 384), lambda b: (b, 0, 0)),
                   pl.BlockSpec((1, 6, 6), lambda b: (b, 0, 0))],
        out_shape=[jax.ShapeDtypeStruct((_B, 384, 384), jnp.int32),
                   jax.ShapeDtypeStruct((_B, 6, 6), jnp.int32)],
    )(sinogram)
    table = (binned.reshape(_B, 12, 32, 12, 32)
             .transpose(0, 1, 3, 2, 4).reshape(_B * 144, 1024))
    pad = ((jnp.arange(_B, dtype=jnp.int32) * 144)[:, None]
           + jnp.arange(_NPAD - _NSEL, dtype=jnp.int32)[None, :])
    best48 = jnp.concatenate([best.reshape(_B, _NSEL), pad], axis=1)
    mi = _make_sc_mi()(jnp.asarray(_FTAB),
                       jnp.asarray(_TGT48), best48, table)
    return mi.reshape(_B, _NPAD)[:, :_NSEL]


# speedup vs baseline: 12.2376x; 1.1060x over previous
import functools

import numpy as np
import jax
import jax.numpy as jnp
from jax import lax
from jax.experimental import pallas as pl
from jax.experimental.pallas import tpu as pltpu
from jax.experimental.pallas import tpu_sc as plsc

_B = 32
_NSEL = 36
_NPAD = 48
_TBL = 1040


def _cl(v, lo, hi):
    return max(lo, min(hi, v))


def _tc_body(x_ref, bin_ref, best_ref):
    x = x_ref[0, 0]
    mn = jnp.min(x)
    mx = jnp.max(x)
    norm = jnp.clip((x - mn) / (mx - mn + 1e-6), 0.0, 1.0)
    bi = jnp.clip((norm * 63.0).astype(jnp.int32), 0, 63)
    bin_ref[0] = bi
    bf = bi.astype(jnp.float32)
    trows = jnp.concatenate([bf[64 * k:64 * k + 32] for k in range(6)], axis=0)
    tsel = jnp.concatenate(
        [trows[:, 64 * l:64 * l + 32] for l in range(6)], axis=1)
    r6 = lax.broadcasted_iota(jnp.int32, (6, 192), 0)
    j192 = lax.broadcasted_iota(jnp.int32, (6, 192), 1)
    srow = (j192 // 32 == r6).astype(jnp.float32)
    jc = lax.broadcasted_iota(jnp.int32, (192, 6), 0)
    c6 = lax.broadcasted_iota(jnp.int32, (192, 6), 1)
    scol = (jc // 32 == c6).astype(jnp.float32)
    kk = lax.broadcasted_iota(jnp.int32, (6, 6), 0)
    ll = lax.broadcasted_iota(jnp.int32, (6, 6), 1)
    best_d = None
    best_p = None
    for dr in range(-2, 3):
        rblocks = [_cl(2 * k + dr, 0, 11) for k in range(6)]
        rsel = jnp.concatenate(
            [bf[32 * rb:32 * rb + 32] for rb in rblocks], axis=0)
        for dc in range(-2, 3):
            cblocks = [_cl(2 * l + dc, 0, 11) for l in range(6)]
            csel = jnp.concatenate(
                [rsel[:, 32 * cb:32 * cb + 32] for cb in cblocks], axis=1)
            d = jnp.abs(tsel - csel)
            dsum = jnp.dot(srow, d, preferred_element_type=jnp.float32)
            dist = jnp.dot(dsum, scol, preferred_element_type=jnp.float32)
            if dr == 0:
                rhit = kk >= 0
            elif dr < 0:
                rhit = kk == 0
            else:
                rhit = kk < 0
            if dc == 0:
                chit = ll >= 0
            elif dc < 0:
                chit = ll == 0
            else:
                chit = ll < 0
            mask = jnp.where(rhit & chit, 1e9, 0.0).astype(jnp.float32)
            dist = dist + mask
            bpj = (jnp.clip(2 * kk + dr, 0, 11) * 12
                   + jnp.clip(2 * ll + dc, 0, 11))
            if best_d is None:
                best_d, best_p = dist, bpj
            else:
                upd = dist < best_d
                best_p = jnp.where(upd, bpj, best_p)
                best_d = jnp.minimum(best_d, dist)
    b = pl.program_id(0)
    best_ref[0] = best_p + b * 144


@functools.cache
def _make_sc_mi():
    mesh = plsc.VectorSubcoreMesh(core_axis_name="c", subcore_axis_name="s")

    @functools.partial(
        pl.kernel,
        mesh=mesh,
        compiler_params=pltpu.CompilerParams(needs_layout_passes=False),
        out_type=jax.ShapeDtypeStruct((_B * _NPAD,), jnp.float32),
        scratch_types=[
            pltpu.VMEM((_NPAD,), jnp.int32),
            pltpu.VMEM((_NPAD,), jnp.int32),
            pltpu.VMEM((32,), jnp.int32),
            pltpu.VMEM((32, 1024), jnp.int32),
            pltpu.VMEM((65536,), jnp.int32),
            pltpu.VMEM((1024,), jnp.int32),
            pltpu.VMEM((1024,), jnp.int32),
            pltpu.VMEM((_TBL * 16,), jnp.float32),
            pltpu.VMEM((_NPAD,), jnp.float32),
            pltpu.SemaphoreType.DMA,
        ],
    )
    def sc_mi(ftab_h, tgt_h, best_h, table_h, out_h,
              ids_t, ids_b, idx_buf, rows_v, histj, hx, hy,
              ftab_v, out_v, sem):
        wid = lax.axis_index("s") * 2 + lax.axis_index("c")
        iota = lax.iota(jnp.int32, 16)
        ones = jnp.ones((16,), jnp.int32)
        zi = jnp.zeros((16,), jnp.int32)
        zf = jnp.zeros((16,), jnp.float32)
        pltpu.sync_copy(ftab_h, ftab_v)
        pltpu.sync_copy(tgt_h.at[wid], ids_t)
        pltpu.sync_copy(best_h.at[wid], ids_b)

        def zj(i, c):
            b0 = i * 64
            histj[pl.ds(b0, 16)] = zi
            histj[pl.ds(b0 + 16, 16)] = zi
            histj[pl.ds(b0 + 32, 16)] = zi
            histj[pl.ds(b0 + 48, 16)] = zi
            return c
        lax.fori_loop(0, 1024, zj, 0)

        def zxy(i, c):
            hx[pl.ds(i * 16, 16)] = zi
            hy[pl.ds(i * 16, 16)] = zi
            return c
        lax.fori_loop(0, 64, zxy, 0)

        for g in range(3):
            idx_buf[pl.ds(0, 16)] = ids_t[pl.ds(g * 16, 16)]
            idx_buf[pl.ds(16, 16)] = ids_b[pl.ds(g * 16, 16)]
            pltpu.async_copy(table_h.at[idx_buf], rows_v, sem).wait()

            def p1(it, c):
                for u in range(8):
                    p = it * 8 + u
                    pp = (iota + p) & 1023
                    xr = plsc.load_gather(rows_v, [iota, pp])
                    yr = plsc.load_gather(rows_v, [iota + 16, pp])
                    jx = (xr * 64 + yr) * 16 + iota
                    plsc.addupdate_scatter(histj, [jx], ones)
                    plsc.addupdate_scatter(hx, [xr * 16 + iota], ones)
                    plsc.addupdate_scatter(hy, [yr * 16 + iota], ones)
                return c
            lax.fori_loop(0, 128, p1, 0)

            def p2(it, acc):
                for u in range(8):
                    p = it * 8 + u
                    pp = (iota + p) & 1023
                    xr = plsc.load_gather(rows_v, [iota, pp])
                    yr = plsc.load_gather(rows_v, [iota + 16, pp])
                    jx = (xr * 64 + yr) * 16 + iota
                    cnt = plsc.load_gather(histj, [jx])
                    plsc.store_scatter(histj, [jx], zi)
                    acc = acc + plsc.load_gather(ftab_v, [cnt * 16 + iota])
                return acc
            accj = lax.fori_loop(0, 128, p2, zf)

            def pxy(a, accs):
                ax, ay = accs
                cx = hx[pl.ds(a * 16, 16)]
                cy = hy[pl.ds(a * 16, 16)]
                ax = ax + plsc.load_gather(ftab_v, [cx * 16 + iota])
                ay = ay + plsc.load_gather(ftab_v, [cy * 16 + iota])
                hx[pl.ds(a * 16, 16)] = zi
                hy[pl.ds(a * 16, 16)] = zi
                return (ax, ay)
            accx, accy = lax.fori_loop(0, 64, pxy, (zf, zf))

            mi = accj - accx - accy
            v = mi + 1.0
            bits = plsc.bitcast(v, jnp.int32)
            e = (bits >> 23) - 127
            m = plsc.bitcast((bits & 0x7FFFFF) | 0x3F800000, jnp.float32)
            t = (m - 1.0) / (m + 1.0)
            t2 = t * t
            ln_m = t * (2.0 + t2 * (0.6666666666 + t2 * (
                0.4 + t2 * (0.2857142857 + t2 * 0.2222222222))))
            res = e.astype(jnp.float32) * 0.6931471805599453 + ln_m
            out_v[pl.ds(g * 16, 16)] = res

        pltpu.sync_copy(out_v, out_h.at[pl.ds(wid * _NPAD, _NPAD)])

    return sc_mi


def _tables():
    c = np.arange(_TBL, dtype=np.float64)
    p = c / 1024.0
    f = p * np.log(p + 1e-8)
    return np.repeat(f.astype(np.float32), 16)


_FTAB = _tables()


def _tgt_ids():
    ids = np.zeros((_B, _NPAD), np.int32)
    for b in range(_B):
        for t in range(_NSEL):
            ids[b, t] = b * 144 + (2 * (t // 6)) * 12 + 2 * (t % 6)
        ids[b, _NSEL:] = b * 144 + np.arange(_NPAD - _NSEL)
    return ids


_TGT48 = _tgt_ids()


def kernel(sinogram):
    if sinogram.ndim == 3:
        sinogram = sinogram[:, None]
    binned, best = pl.pallas_call(
        _tc_body,
        grid=(_B,),
        in_specs=[pl.BlockSpec((1, 1, 384, 384), lambda b: (b, 0, 0, 0))],
        out_specs=[pl.BlockSpec((1, 384, 384), lambda b: (b, 0, 0)),
                   pl.BlockSpec((1, 6, 6), lambda b: (b, 0, 0))],
        out_shape=[jax.ShapeDtypeStruct((_B, 384, 384), jnp.int32),
                   jax.ShapeDtypeStruct((_B, 6, 6), jnp.int32)],
    )(sinogram)
    table = (binned.reshape(_B, 12, 32, 12, 32)
             .transpose(0, 1, 3, 2, 4).reshape(_B * 144, 1024))
    pad = ((jnp.arange(_B, dtype=jnp.int32) * 144)[:, None]
           + jnp.arange(_NPAD - _NSEL, dtype=jnp.int32)[None, :])
    best48 = jnp.concatenate([best.reshape(_B, _NSEL), pad], axis=1)
    mi = _make_sc_mi()(jnp.asarray(_FTAB),
                       jnp.asarray(_TGT48), best48, table)
    return mi.reshape(_B, _NPAD)[:, :_NSEL]


# speedup vs baseline: 23.7983x; 1.9447x over previous
import functools

import numpy as np
import jax
import jax.numpy as jnp
from jax import lax
from jax.experimental import pallas as pl
from jax.experimental.pallas import tpu as pltpu
from jax.experimental.pallas import tpu_sc as plsc

_B = 32
_NSEL = 36
_NPAD = 48
_TBL = 1040


def _cl(v, lo, hi):
    return max(lo, min(hi, v))


def _tc_body(x_ref, bin_ref, best_ref):
    x = x_ref[0, 0]
    mn = jnp.min(x)
    mx = jnp.max(x)
    norm = jnp.clip((x - mn) / (mx - mn + 1e-6), 0.0, 1.0)
    bi = jnp.clip((norm * 63.0).astype(jnp.int32), 0, 63)
    bin_ref[0] = (bi.reshape(12, 32, 12, 32).transpose(0, 2, 1, 3)
                  .reshape(144, 1024))
    bf = bi.astype(jnp.float32)
    trows = jnp.concatenate([bf[64 * k:64 * k + 32] for k in range(6)], axis=0)
    tsel = jnp.concatenate(
        [trows[:, 64 * l:64 * l + 32] for l in range(6)], axis=1)
    r6 = lax.broadcasted_iota(jnp.int32, (6, 192), 0)
    j192 = lax.broadcasted_iota(jnp.int32, (6, 192), 1)
    srow = (j192 // 32 == r6).astype(jnp.float32)
    jc = lax.broadcasted_iota(jnp.int32, (192, 6), 0)
    c6 = lax.broadcasted_iota(jnp.int32, (192, 6), 1)
    scol = (jc // 32 == c6).astype(jnp.float32)
    kk = lax.broadcasted_iota(jnp.int32, (6, 6), 0)
    ll = lax.broadcasted_iota(jnp.int32, (6, 6), 1)
    best_d = None
    best_p = None
    for dr in range(-2, 3):
        rblocks = [_cl(2 * k + dr, 0, 11) for k in range(6)]
        rsel = jnp.concatenate(
            [bf[32 * rb:32 * rb + 32] for rb in rblocks], axis=0)
        for dc in range(-2, 3):
            cblocks = [_cl(2 * l + dc, 0, 11) for l in range(6)]
            csel = jnp.concatenate(
                [rsel[:, 32 * cb:32 * cb + 32] for cb in cblocks], axis=1)
            d = jnp.abs(tsel - csel)
            dsum = jnp.dot(srow, d, preferred_element_type=jnp.float32)
            dist = jnp.dot(dsum, scol, preferred_element_type=jnp.float32)
            if dr == 0:
                rhit = kk >= 0
            elif dr < 0:
                rhit = kk == 0
            else:
                rhit = kk < 0
            if dc == 0:
                chit = ll >= 0
            elif dc < 0:
                chit = ll == 0
            else:
                chit = ll < 0
            mask = jnp.where(rhit & chit, 1e9, 0.0).astype(jnp.float32)
            dist = dist + mask
            bpj = (jnp.clip(2 * kk + dr, 0, 11) * 12
                   + jnp.clip(2 * ll + dc, 0, 11))
            if best_d is None:
                best_d, best_p = dist, bpj
            else:
                upd = dist < best_d
                best_p = jnp.where(upd, bpj, best_p)
                best_d = jnp.minimum(best_d, dist)
    b = pl.program_id(0)
    best_ref[0] = best_p + b * 144


@functools.cache
def _make_sc_mi():
    mesh = plsc.VectorSubcoreMesh(core_axis_name="c", subcore_axis_name="s")

    @functools.partial(
        pl.kernel,
        mesh=mesh,
        compiler_params=pltpu.CompilerParams(needs_layout_passes=False,
                                             use_tc_tiling_on_sc=True),
        out_type=jax.ShapeDtypeStruct((_B * _NPAD,), jnp.float32),
        scratch_types=[
            pltpu.VMEM((_NPAD,), jnp.int32),
            pltpu.VMEM((_NPAD,), jnp.int32),
            pltpu.VMEM((32,), jnp.int32),
            pltpu.VMEM((32, 1024), jnp.int32),
            pltpu.VMEM((8192,), jnp.int32),
            pltpu.VMEM((65536,), jnp.int32),
            pltpu.VMEM((1024,), jnp.int32),
            pltpu.VMEM((1024,), jnp.int32),
            pltpu.VMEM((_TBL * 16,), jnp.float32),
            pltpu.VMEM((_NPAD,), jnp.float32),
            pltpu.SemaphoreType.DMA,
        ],
    )
    def sc_mi(ftab_h, tgt_h, best_h, table_h, out_h,
              ids_t, ids_b, idx_buf, rows_v, jbuf, histj, hx, hy,
              ftab_v, out_v, sem):
        wid = lax.axis_index("s") * 2 + lax.axis_index("c")
        iota = lax.iota(jnp.int32, 16)
        ones = jnp.ones((16,), jnp.int32)
        zi = jnp.zeros((16,), jnp.int32)
        zf = jnp.zeros((16,), jnp.float32)
        pltpu.sync_copy(ftab_h, ftab_v)
        pltpu.sync_copy(tgt_h.at[wid], ids_t)
        pltpu.sync_copy(best_h.at[wid], ids_b)

        def zj(i, c):
            b0 = i * 64
            histj[pl.ds(b0, 16)] = zi
            histj[pl.ds(b0 + 16, 16)] = zi
            histj[pl.ds(b0 + 32, 16)] = zi
            histj[pl.ds(b0 + 48, 16)] = zi
            return c
        lax.fori_loop(0, 1024, zj, 0)

        def zxy(i, c):
            hx[pl.ds(i * 16, 16)] = zi
            hy[pl.ds(i * 16, 16)] = zi
            return c
        lax.fori_loop(0, 64, zxy, 0)

        for g in range(3):
            idx_buf[pl.ds(0, 16)] = ids_t[pl.ds(g * 16, 16)]
            idx_buf[pl.ds(16, 16)] = ids_b[pl.ds(g * 16, 16)]
            pltpu.async_copy(table_h.at[idx_buf], rows_v, sem).wait()

            @plsc.parallel_loop(0, 128, step=1, unroll=2)
            def _p1(it):
                jxs = []
                for u in range(8):
                    p = it * 8 + u
                    pp = (iota + p) & 1023
                    xr = plsc.load_gather(rows_v, [iota, pp])
                    yr = plsc.load_gather(rows_v, [iota + 16, pp])
                    t1 = yr * 16 + iota
                    jx = xr * 1024 + t1
                    jxs.append(jx)
                    plsc.addupdate_scatter(histj, [jx], ones)
                    plsc.addupdate_scatter(hx, [xr * 16 + iota], ones)
                    plsc.addupdate_scatter(hy, [t1], ones)
                for u in range(4):
                    q = it * 4 + u
                    jbuf[pl.ds(q * 16, 16)] = (jxs[2 * u]
                                               | (jxs[2 * u + 1] << 16))

            def p2(it, acc):
                for u in range(4):
                    q = it * 4 + u
                    pk = jbuf[pl.ds(q * 16, 16)]
                    for jx in (pk & 0xFFFF,
                               lax.shift_right_logical(pk, 16)):
                        cnt = plsc.load_gather(histj, [jx])
                        plsc.store_scatter(histj, [jx], zi)
                        acc = acc + plsc.load_gather(ftab_v,
                                                     [cnt * 16 + iota])
                return acc
            accj = lax.fori_loop(0, 128, p2, zf)

            def pxy(a, accs):
                ax, ay = accs
                cx = hx[pl.ds(a * 16, 16)]
                cy = hy[pl.ds(a * 16, 16)]
                ax = ax + plsc.load_gather(ftab_v, [cx * 16 + iota])
                ay = ay + plsc.load_gather(ftab_v, [cy * 16 + iota])
                hx[pl.ds(a * 16, 16)] = zi
                hy[pl.ds(a * 16, 16)] = zi
                return (ax, ay)
            accx, accy = lax.fori_loop(0, 64, pxy, (zf, zf))

            mi = accj - accx - accy
            v = mi + 1.0
            bits = plsc.bitcast(v, jnp.int32)
            e = (bits >> 23) - 127
            m = plsc.bitcast((bits & 0x7FFFFF) | 0x3F800000, jnp.float32)
            t = (m - 1.0) / (m + 1.0)
            t2 = t * t
            ln_m = t * (2.0 + t2 * (0.6666666666 + t2 * (
                0.4 + t2 * (0.2857142857 + t2 * 0.2222222222))))
            res = e.astype(jnp.float32) * 0.6931471805599453 + ln_m
            out_v[pl.ds(g * 16, 16)] = res

        pltpu.sync_copy(out_v, out_h.at[pl.ds(wid * _NPAD, _NPAD)])

    return sc_mi


def _tables():
    c = np.arange(_TBL, dtype=np.float64)
    p = c / 1024.0
    f = p * np.log(p + 1e-8)
    return np.repeat(f.astype(np.float32), 16)


_FTAB = _tables()


def _tgt_ids():
    ids = np.zeros((_B, _NPAD), np.int32)
    for b in range(_B):
        for t in range(_NSEL):
            ids[b, t] = b * 144 + (2 * (t // 6)) * 12 + 2 * (t % 6)
        ids[b, _NSEL:] = b * 144 + np.arange(_NPAD - _NSEL)
    return ids


_TGT48 = _tgt_ids()


def kernel(sinogram):
    if sinogram.ndim == 3:
        sinogram = sinogram[:, None]
    binned, best = pl.pallas_call(
        _tc_body,
        grid=(_B,),
        in_specs=[pl.BlockSpec((1, 1, 384, 384), lambda b: (b, 0, 0, 0))],
        out_specs=[pl.BlockSpec((1, 144, 1024), lambda b: (b, 0, 0)),
                   pl.BlockSpec((1, 6, 6), lambda b: (b, 0, 0))],
        out_shape=[jax.ShapeDtypeStruct((_B, 144, 1024), jnp.int32),
                   jax.ShapeDtypeStruct((_B, 6, 6), jnp.int32)],
    )(sinogram)
    table = binned.reshape(_B * 144, 1024)
    pad = ((jnp.arange(_B, dtype=jnp.int32) * 144)[:, None]
           + jnp.arange(_NPAD - _NSEL, dtype=jnp.int32)[None, :])
    best48 = jnp.concatenate([best.reshape(_B, _NSEL), pad], axis=1)
    mi = _make_sc_mi()(jnp.asarray(_FTAB),
                       jnp.asarray(_TGT48), best48, table)
    return mi.reshape(_B, _NPAD)[:, :_NSEL]


# speedup vs baseline: 26.3125x; 1.1056x over previous
import functools

import numpy as np
import jax
import jax.numpy as jnp
from jax import lax
from jax.experimental import pallas as pl
from jax.experimental.pallas import tpu as pltpu
from jax.experimental.pallas import tpu_sc as plsc

_B = 32
_NSEL = 36
_NPAD = 48
_TBL = 1040


def _cl(v, lo, hi):
    return max(lo, min(hi, v))


def _tc_body(x_ref, bin_ref, best_ref):
    x = x_ref[0, 0]
    mn = jnp.min(x)
    mx = jnp.max(x)
    norm = jnp.clip((x - mn) / (mx - mn + 1e-6), 0.0, 1.0)
    bi = jnp.clip((norm * 63.0).astype(jnp.int32), 0, 63)
    bin_ref[0] = (bi.reshape(12, 32, 12, 32).transpose(0, 2, 1, 3)
                  .reshape(144, 1024))
    bf = bi.astype(jnp.float32)
    trows = jnp.concatenate([bf[64 * k:64 * k + 32] for k in range(6)], axis=0)
    tsel = jnp.concatenate(
        [trows[:, 64 * l:64 * l + 32] for l in range(6)], axis=1)
    r6 = lax.broadcasted_iota(jnp.int32, (6, 192), 0)
    j192 = lax.broadcasted_iota(jnp.int32, (6, 192), 1)
    srow = (j192 // 32 == r6).astype(jnp.float32)
    jc = lax.broadcasted_iota(jnp.int32, (192, 6), 0)
    c6 = lax.broadcasted_iota(jnp.int32, (192, 6), 1)
    scol = (jc // 32 == c6).astype(jnp.float32)
    kk = lax.broadcasted_iota(jnp.int32, (6, 6), 0)
    ll = lax.broadcasted_iota(jnp.int32, (6, 6), 1)
    best_d = None
    best_p = None
    for dr in range(-2, 3):
        rblocks = [_cl(2 * k + dr, 0, 11) for k in range(6)]
        rsel = jnp.concatenate(
            [bf[32 * rb:32 * rb + 32] for rb in rblocks], axis=0)
        for dc in range(-2, 3):
            cblocks = [_cl(2 * l + dc, 0, 11) for l in range(6)]
            csel = jnp.concatenate(
                [rsel[:, 32 * cb:32 * cb + 32] for cb in cblocks], axis=1)
            d = jnp.abs(tsel - csel)
            dsum = jnp.dot(srow, d, preferred_element_type=jnp.float32)
            dist = jnp.dot(dsum, scol, preferred_element_type=jnp.float32)
            if dr == 0:
                rhit = kk >= 0
            elif dr < 0:
                rhit = kk == 0
            else:
                rhit = kk < 0
            if dc == 0:
                chit = ll >= 0
            elif dc < 0:
                chit = ll == 0
            else:
                chit = ll < 0
            mask = jnp.where(rhit & chit, 1e9, 0.0).astype(jnp.float32)
            dist = dist + mask
            bpj = (jnp.clip(2 * kk + dr, 0, 11) * 12
                   + jnp.clip(2 * ll + dc, 0, 11))
            if best_d is None:
                best_d, best_p = dist, bpj
            else:
                upd = dist < best_d
                best_p = jnp.where(upd, bpj, best_p)
                best_d = jnp.minimum(best_d, dist)
    b = pl.program_id(0)
    best_ref[0] = best_p + b * 144


@functools.cache
def _make_sc_mi():
    mesh = plsc.VectorSubcoreMesh(core_axis_name="c", subcore_axis_name="s")

    @functools.partial(
        pl.kernel,
        mesh=mesh,
        compiler_params=pltpu.CompilerParams(needs_layout_passes=False,
                                             use_tc_tiling_on_sc=True),
        out_type=jax.ShapeDtypeStruct((_B * _NPAD,), jnp.float32),
        scratch_types=[
            pltpu.VMEM((_NPAD,), jnp.int32),
            pltpu.VMEM((_NPAD,), jnp.int32),
            pltpu.VMEM((32,), jnp.int32),
            pltpu.VMEM((32, 1024), jnp.int32),
            pltpu.VMEM((8192,), jnp.int32),
            pltpu.VMEM((65536,), jnp.int32),
            pltpu.VMEM((1024,), jnp.int32),
            pltpu.VMEM((1024,), jnp.int32),
            pltpu.VMEM((_TBL,), jnp.float32),
            pltpu.VMEM((_TBL * 16,), jnp.float32),
            pltpu.VMEM((_NPAD,), jnp.float32),
            pltpu.SemaphoreType.DMA,
        ],
    )
    def sc_mi(ftab_h, gtab_h, tgt_h, best_h, table_h, out_h,
              ids_t, ids_b, idx_buf, rows_v, jbuf, histj, hx, hy,
              ftab_v, gtab_v, out_v, sem):
        wid = lax.axis_index("s") * 2 + lax.axis_index("c")
        iota = lax.iota(jnp.int32, 16)
        ones = jnp.ones((16,), jnp.int32)
        zi = jnp.zeros((16,), jnp.int32)
        zf = jnp.zeros((16,), jnp.float32)
        pltpu.sync_copy(ftab_h, ftab_v)
        pltpu.sync_copy(gtab_h, gtab_v)
        pltpu.sync_copy(tgt_h.at[wid], ids_t)
        pltpu.sync_copy(best_h.at[wid], ids_b)

        @plsc.parallel_loop(0, 1024, step=1, unroll=4)
        def _zj(i):
            b0 = i * 64
            histj[pl.ds(b0, 16)] = zi
            histj[pl.ds(b0 + 16, 16)] = zi
            histj[pl.ds(b0 + 32, 16)] = zi
            histj[pl.ds(b0 + 48, 16)] = zi

        @plsc.parallel_loop(0, 64, step=1, unroll=4)
        def _zxy(i):
            hx[pl.ds(i * 16, 16)] = zi
            hy[pl.ds(i * 16, 16)] = zi

        for g in range(3):
            idx_buf[pl.ds(0, 16)] = ids_t[pl.ds(g * 16, 16)]
            idx_buf[pl.ds(16, 16)] = ids_b[pl.ds(g * 16, 16)]
            pltpu.async_copy(table_h.at[idx_buf], rows_v, sem).wait()

            @plsc.parallel_loop(0, 128, step=1, unroll=2)
            def _p1(it):
                jxs = []
                for u in range(8):
                    p = it * 8 + u
                    pp = (iota + p) & 1023
                    xr = plsc.load_gather(rows_v, [iota, pp])
                    yr = plsc.load_gather(rows_v, [iota + 16, pp])
                    t1 = yr * 16 + iota
                    jx = xr * 1024 + t1
                    jxs.append(jx)
                    plsc.addupdate_scatter(histj, [jx], ones)
                    plsc.addupdate_scatter(hx, [xr * 16 + iota], ones)
                    plsc.addupdate_scatter(hy, [t1], ones)
                for u in range(4):
                    q = it * 4 + u
                    jbuf[pl.ds(q * 16, 16)] = (jxs[2 * u]
                                               | (jxs[2 * u + 1] << 16))

            def _p2(it, acc):
                for u in range(4):
                    q = it * 4 + u
                    pk = jbuf[pl.ds(q * 16, 16)]
                    for jx in (pk & 0xFFFF,
                               lax.shift_right_logical(pk, 16)):
                        cnt = plsc.load_gather(histj, [jx])
                        acc = acc + plsc.load_gather(gtab_v,
                                                     [cnt * 16 + iota])
                return acc
            accj = plsc.parallel_loop(0, 128, step=1, unroll=2,
                                      carry=zf)(
                lambda it, acc: _p2(it, acc))

            @plsc.parallel_loop(0, 128, step=1, unroll=2)
            def _p3(it):
                for u in range(4):
                    q = it * 4 + u
                    pk = jbuf[pl.ds(q * 16, 16)]
                    plsc.store_scatter(histj, [pk & 0xFFFF], zi)
                    plsc.store_scatter(
                        histj, [lax.shift_right_logical(pk, 16)], zi)

            def pxy(a, accs):
                ax, ay = accs
                cx = hx[pl.ds(a * 16, 16)]
                cy = hy[pl.ds(a * 16, 16)]
                ax = ax + plsc.load_gather(ftab_v, [cx])
                ay = ay + plsc.load_gather(ftab_v, [cy])
                hx[pl.ds(a * 16, 16)] = zi
                hy[pl.ds(a * 16, 16)] = zi
                return (ax, ay)
            accx, accy = lax.fori_loop(0, 64, pxy, (zf, zf))

            mi = accj - accx - accy
            v = mi + 1.0
            bits = plsc.bitcast(v, jnp.int32)
            e = (bits >> 23) - 127
            m = plsc.bitcast((bits & 0x7FFFFF) | 0x3F800000, jnp.float32)
            t = (m - 1.0) / (m + 1.0)
            t2 = t * t
            ln_m = t * (2.0 + t2 * (0.6666666666 + t2 * (
                0.4 + t2 * (0.2857142857 + t2 * 0.2222222222))))
            res = e.astype(jnp.float32) * 0.6931471805599453 + ln_m
            out_v[pl.ds(g * 16, 16)] = res

        pltpu.sync_copy(out_v, out_h.at[pl.ds(wid * _NPAD, _NPAD)])

    return sc_mi


def _tables():
    c = np.arange(_TBL, dtype=np.float64)
    p = c / 1024.0
    f = p * np.log(p + 1e-8)
    g = np.zeros(_TBL, np.float64)
    g[1:] = f[1:] / c[1:]
    return f.astype(np.float32), np.repeat(g.astype(np.float32), 16)


_FTAB, _GTAB = _tables()


def _tgt_ids():
    ids = np.zeros((_B, _NPAD), np.int32)
    for b in range(_B):
        for t in range(_NSEL):
            ids[b, t] = b * 144 + (2 * (t // 6)) * 12 + 2 * (t % 6)
        ids[b, _NSEL:] = b * 144 + np.arange(_NPAD - _NSEL)
    return ids


_TGT48 = _tgt_ids()


def kernel(sinogram):
    if sinogram.ndim == 3:
        sinogram = sinogram[:, None]
    binned, best = pl.pallas_call(
        _tc_body,
        grid=(_B,),
        in_specs=[pl.BlockSpec((1, 1, 384, 384), lambda b: (b, 0, 0, 0))],
        out_specs=[pl.BlockSpec((1, 144, 1024), lambda b: (b, 0, 0)),
                   pl.BlockSpec((1, 6, 6), lambda b: (b, 0, 0))],
        out_shape=[jax.ShapeDtypeStruct((_B, 144, 1024), jnp.int32),
                   jax.ShapeDtypeStruct((_B, 6, 6), jnp.int32)],
    )(sinogram)
    table = binned.reshape(_B * 144, 1024)
    pad = ((jnp.arange(_B, dtype=jnp.int32) * 144)[:, None]
           + jnp.arange(_NPAD - _NSEL, dtype=jnp.int32)[None, :])
    best48 = jnp.concatenate([best.reshape(_B, _NSEL), pad], axis=1)
    mi = _make_sc_mi()(jnp.asarray(_FTAB), jnp.asarray(_GTAB),
                       jnp.asarray(_TGT48), best48, table)
    return mi.reshape(_B, _NPAD)[:, :_NSEL]
